# baseline (speedup 1.0000x reference)
import functools

import jax
import jax.numpy as jnp
from jax import lax
from jax.experimental import pallas as pl
from jax.experimental.pallas import tpu as pltpu
from jax.experimental.pallas import tpu_sc as plsc

NC = 2
NS = 16
NW = NC * NS

GW = 128
NBUF = 4


def _sc_gather(B, VR, D):
    rows_total = B // GW
    rpw = rows_total // NW
    LOAD = ((rpw + 6 + 7) // 8) * 8
    assert rpw * NW == rows_total

    mesh = plsc.VectorSubcoreMesh(
        core_axis_name="c", subcore_axis_name="s", num_cores=NC, num_subcores=NS
    )

    @functools.partial(
        pl.kernel,
        mesh=mesh,
        out_type=jax.ShapeDtypeStruct((B, D), jnp.float32),
        scratch_types=[
            pltpu.VMEM((LOAD, GW), jnp.int32),
            pltpu.VMEM((NBUF, GW, D), jnp.float32),
            [pltpu.SemaphoreType.DMA] * NBUF,
            [pltpu.SemaphoreType.DMA] * NBUF,
        ],
        compiler_params=pltpu.CompilerParams(use_tc_tiling_on_sc=True),
    )
    def k(idx_hbm, tok_hbm, out_hbm, idx_v, buf_v, gsem, wsem):
        wid = lax.axis_index("s") * NC + lax.axis_index("c")
        row_start = wid * rpw
        r0 = pl.multiple_of((row_start // 8) * 8, 8)
        lo = row_start - r0
        pltpu.sync_copy(idx_hbm.at[pl.ds(r0, LOAD)], idx_v)

        gd = {}
        wd = {}
        lag = NBUF - 1
        for j in range(rpw + lag):
            if j < rpw:
                if j >= NBUF:
                    wd[j - NBUF].wait()
                gd[j] = pltpu.async_copy(
                    tok_hbm.at[idx_v.at[lo + j]], buf_v.at[j % NBUF], gsem[j % NBUF]
                )
            if j >= lag:
                jj = j - lag
                gd[jj].wait()
                wd[jj] = pltpu.async_copy(
                    buf_v.at[jj % NBUF],
                    out_hbm.at[pl.ds(pl.multiple_of((row_start + jj) * GW, GW), GW)],
                    wsem[jj % NBUF],
                )
        for jj in range(max(rpw - NBUF, 0), rpw):
            wd[jj].wait()

    return k


KW = 8192
HB = (KW // 2).bit_length() - 1


def _tc_transpose(V, D):
    nblk = (V + KW - 1) // KW
    vrows = nblk * (KW // 2)

    def body(t_ref, out_ref):
        x = t_ref[...]
        a = jnp.transpose(x[:, : KW // 2], (1, 0))
        b = jnp.transpose(x[:, KW // 2 :], (1, 0))
        out_ref[...] = jnp.concatenate([a, b], axis=1)

    return pl.pallas_call(
        body,
        grid=(nblk,),
        in_specs=[pl.BlockSpec((D, KW), lambda m: (0, m))],
        out_specs=pl.BlockSpec((KW // 2, 2 * D), lambda m: (m, 0)),
        out_shape=jax.ShapeDtypeStruct((vrows, 2 * D), jnp.float32),
    )


def _tc_fixup(Bt, T, D):
    TB = 40
    BB = 256

    def body(g_ref, idx_ref, pos_ref, out_ref):
        x = g_ref[...]
        ti = pl.program_id(1)
        idxb = idx_ref[pl.ds(ti * TB, TB), :]
        par = ((idxb >> HB) & 1) == 1
        for t in range(TB):
            xt = jnp.transpose(x[:, t, :], (1, 0))
            sel = jnp.where(par[t:t + 1, :], xt[D:], xt[:D])
            out_ref[t] = sel + pos_ref[t][:, None]

    return pl.pallas_call(
        body,
        grid=(Bt // BB, T // TB),
        in_specs=[
            pl.BlockSpec((BB, TB, 2 * D), lambda bi, ti: (bi, ti, 0)),
            pl.BlockSpec((T, BB), lambda bi, ti: (0, bi)),
            pl.BlockSpec((TB, D), lambda bi, ti: (ti, 0)),
        ],
        out_specs=pl.BlockSpec((TB, D, BB), lambda bi, ti: (ti, 0, bi)),
        out_shape=jax.ShapeDtypeStruct((T, D, Bt), jnp.float32),
    )


def kernel(inputs, token_table, pos_table):
    Bt, T = inputs.shape
    V, D = token_table.shape
    B = Bt * T
    idx32 = inputs.astype(jnp.int32)
    kh = KW // 2
    idx_rows = jnp.reshape(
        (idx32 // KW) * kh + (idx32 % kh), (B // GW, GW)
    )
    tbl = _tc_transpose(V, D)(jnp.transpose(token_table))
    gathered = _sc_gather(B, V // 2, 2 * D)(idx_rows, tbl)
    g3 = jnp.reshape(gathered, (Bt, T, 2 * D))
    out_t = _tc_fixup(Bt, T, D)(g3, jnp.transpose(idx32), pos_table)
    return jnp.transpose(out_t, (2, 0, 1))


# speedup vs baseline: 1.0571x; 1.0571x over previous
import functools

import jax
import jax.numpy as jnp
from jax import lax
from jax.experimental import pallas as pl
from jax.experimental.pallas import tpu as pltpu
from jax.experimental.pallas import tpu_sc as plsc

NC = 2
NS = 16
NW = NC * NS

GW = 128
NBUF = 4


def _sc_gather(B, VR, D):
    rows_total = B // GW
    rpw = rows_total // NW
    LOAD = ((rpw + 6 + 7) // 8) * 8
    assert rpw * NW == rows_total

    mesh = plsc.VectorSubcoreMesh(
        core_axis_name="c", subcore_axis_name="s", num_cores=NC, num_subcores=NS
    )

    @functools.partial(
        pl.kernel,
        mesh=mesh,
        out_type=jax.ShapeDtypeStruct((B, D), jnp.float32),
        scratch_types=[
            pltpu.VMEM((LOAD, GW), jnp.int32),
            pltpu.VMEM((NBUF, GW, D), jnp.float32),
            [pltpu.SemaphoreType.DMA] * NBUF,
            [pltpu.SemaphoreType.DMA] * NBUF,
        ],
        compiler_params=pltpu.CompilerParams(use_tc_tiling_on_sc=True),
    )
    def k(idx_hbm, tok_hbm, out_hbm, idx_v, buf_v, gsem, wsem):
        wid = lax.axis_index("s") * NC + lax.axis_index("c")
        row_start = wid * rpw
        r0 = pl.multiple_of((row_start // 8) * 8, 8)
        lo = row_start - r0
        pltpu.sync_copy(idx_hbm.at[pl.ds(r0, LOAD)], idx_v)

        gd = {}
        wd = {}
        lag = NBUF - 1
        for j in range(rpw + lag):
            if j < rpw:
                if j >= NBUF:
                    wd[j - NBUF].wait()
                gd[j] = pltpu.async_copy(
                    tok_hbm.at[idx_v.at[lo + j]], buf_v.at[j % NBUF], gsem[j % NBUF]
                )
            if j >= lag:
                jj = j - lag
                gd[jj].wait()
                wd[jj] = pltpu.async_copy(
                    buf_v.at[jj % NBUF],
                    out_hbm.at[pl.ds(pl.multiple_of((row_start + jj) * GW, GW), GW)],
                    wsem[jj % NBUF],
                )
        for jj in range(max(rpw - NBUF, 0), rpw):
            wd[jj].wait()

    return k


KW = 8192
HB = (KW // 2).bit_length() - 1


def _tc_transpose(V, D):
    nblk = (V + KW - 1) // KW
    vrows = nblk * (KW // 2)

    def body(t_ref, out_ref):
        x = t_ref[...]
        a = jnp.transpose(x[:, : KW // 2], (1, 0))
        b = jnp.transpose(x[:, KW // 2 :], (1, 0))
        out_ref[...] = jnp.concatenate([a, b], axis=1)

    return pl.pallas_call(
        body,
        grid=(nblk,),
        in_specs=[pl.BlockSpec((D, KW), lambda m: (0, m))],
        out_specs=pl.BlockSpec((KW // 2, 2 * D), lambda m: (m, 0)),
        out_shape=jax.ShapeDtypeStruct((vrows, 2 * D), jnp.float32),
    )


def _tc_fixup(Bt, T, D, half, nhalf):
    TB = 40
    BB = 256
    bh = Bt // nhalf // BB
    boff = half * bh

    def body(g_ref, idx_ref, pos_ref, *refs):
        out_ref = refs[-1]
        x = g_ref[...]
        ti = pl.program_id(1)
        idxb = idx_ref[pl.ds(ti * TB, TB), :]
        par = ((idxb >> HB) & 1) == 1
        for t in range(TB):
            xt = jnp.transpose(x[:, t, :], (1, 0))
            sel = jnp.where(par[t:t + 1, :], xt[D:], xt[:D])
            out_ref[t] = sel + pos_ref[t][:, None]

    in_specs = [
        pl.BlockSpec((BB, TB, 2 * D), lambda bi, ti: (bi, ti, 0)),
        pl.BlockSpec((T, BB), lambda bi, ti: (0, bi + boff)),
        pl.BlockSpec((TB, D), lambda bi, ti: (ti, 0)),
    ]
    aliases = {}
    if half > 0:
        in_specs.append(pl.BlockSpec(memory_space=pltpu.MemorySpace.HBM))
        aliases = {3: 0}
    return pl.pallas_call(
        body,
        grid=(bh, T // TB),
        in_specs=in_specs,
        out_specs=pl.BlockSpec((TB, D, BB), lambda bi, ti: (ti, 0, bi + boff)),
        out_shape=jax.ShapeDtypeStruct((T, D, Bt), jnp.float32),
        input_output_aliases=aliases,
    )


def kernel(inputs, token_table, pos_table):
    Bt, T = inputs.shape
    V, D = token_table.shape
    B = Bt * T
    idx32 = inputs.astype(jnp.int32)
    kh = KW // 2
    idx_rows = jnp.reshape(
        (idx32 // KW) * kh + (idx32 % kh), (B // GW, GW)
    )
    tbl = _tc_transpose(V, D)(jnp.transpose(token_table))
    idxT = jnp.transpose(idx32)
    NH = 2
    Bh = B // NH
    rows_h = Bh // GW
    out_t = None
    for h in range(NH):
        g = _sc_gather(Bh, V // 2, 2 * D)(
            idx_rows[h * rows_h:(h + 1) * rows_h], tbl
        )
        g3 = jnp.reshape(g, (Bt // NH, T, 2 * D))
        fix = _tc_fixup(Bt, T, D, h, NH)
        if h == 0:
            out_t = fix(g3, idxT, pos_table)
        else:
            out_t = fix(g3, idxT, pos_table, out_t)
    return jnp.transpose(out_t, (2, 0, 1))


# speedup vs baseline: 1.1357x; 1.0744x over previous
import functools

import jax
import jax.numpy as jnp
from jax import lax
from jax.experimental import pallas as pl
from jax.experimental.pallas import tpu as pltpu
from jax.experimental.pallas import tpu_sc as plsc

NC = 2
NS = 16
NW = NC * NS

GW = 128
NBUF = 4


def _sc_gather(B, VR, D):
    rows_total = B // GW
    rpw = rows_total // NW
    LOAD = ((rpw + 6 + 7) // 8) * 8
    assert rpw * NW == rows_total

    mesh = plsc.VectorSubcoreMesh(
        core_axis_name="c", subcore_axis_name="s", num_cores=NC, num_subcores=NS
    )

    @functools.partial(
        pl.kernel,
        mesh=mesh,
        out_type=jax.ShapeDtypeStruct((B, D), jnp.float32),
        scratch_types=[
            pltpu.VMEM((LOAD, GW), jnp.int32),
            pltpu.VMEM((NBUF, GW, D), jnp.float32),
            [pltpu.SemaphoreType.DMA] * NBUF,
            [pltpu.SemaphoreType.DMA] * NBUF,
        ],
        compiler_params=pltpu.CompilerParams(use_tc_tiling_on_sc=True),
    )
    def k(idx_hbm, tok_hbm, out_hbm, idx_v, buf_v, gsem, wsem):
        wid = lax.axis_index("s") * NC + lax.axis_index("c")
        row_start = wid * rpw
        r0 = pl.multiple_of((row_start // 8) * 8, 8)
        lo = row_start - r0
        pltpu.sync_copy(idx_hbm.at[pl.ds(r0, LOAD)], idx_v)

        gd = {}
        wd = {}
        lag = NBUF - 1
        for j in range(rpw + lag):
            if j < rpw:
                if j >= NBUF:
                    wd[j - NBUF].wait()
                gd[j] = pltpu.async_copy(
                    tok_hbm.at[idx_v.at[lo + j]], buf_v.at[j % NBUF], gsem[j % NBUF]
                )
            if j >= lag:
                jj = j - lag
                gd[jj].wait()
                wd[jj] = pltpu.async_copy(
                    buf_v.at[jj % NBUF],
                    out_hbm.at[pl.ds(pl.multiple_of((row_start + jj) * GW, GW), GW)],
                    wsem[jj % NBUF],
                )
        for jj in range(max(rpw - NBUF, 0), rpw):
            wd[jj].wait()

    return k


KW = 16384
HB = (KW // 2).bit_length() - 1


def _tc_transpose(V, D):
    nblk = (V + KW - 1) // KW
    vrows = nblk * (KW // 2)

    def body(t_ref, out_ref):
        x = t_ref[...]
        a = jnp.transpose(x[:, : KW // 2], (1, 0))
        b = jnp.transpose(x[:, KW // 2 :], (1, 0))
        out_ref[...] = jnp.concatenate([a, b], axis=1)

    return pl.pallas_call(
        body,
        grid=(nblk,),
        in_specs=[pl.BlockSpec((D, KW), lambda m: (0, m))],
        out_specs=pl.BlockSpec((KW // 2, 2 * D), lambda m: (m, 0)),
        out_shape=jax.ShapeDtypeStruct((vrows, 2 * D), jnp.float32),
    )


def _tc_fixup(Bt, T, D, half, nhalf):
    TB = 40
    BB = 256
    bh = Bt // nhalf // BB
    boff = half * bh

    def body(g_ref, idx_ref, pos_ref, *refs):
        out_ref = refs[-1]
        x = g_ref[...]
        ti = pl.program_id(1)
        idxb = idx_ref[pl.ds(ti * TB, TB), :]
        par = ((idxb >> HB) & 1) == 1
        for t in range(TB):
            xt = jnp.transpose(x[:, t, :], (1, 0))
            sel = jnp.where(par[t:t + 1, :], xt[D:], xt[:D])
            out_ref[t] = sel + pos_ref[t][:, None]

    in_specs = [
        pl.BlockSpec((BB, TB, 2 * D), lambda bi, ti: (bi, ti, 0)),
        pl.BlockSpec((T, BB), lambda bi, ti: (0, bi + boff)),
        pl.BlockSpec((TB, D), lambda bi, ti: (ti, 0)),
    ]
    aliases = {}
    if half > 0:
        in_specs.append(pl.BlockSpec(memory_space=pltpu.MemorySpace.HBM))
        aliases = {3: 0}
    return pl.pallas_call(
        body,
        grid=(bh, T // TB),
        in_specs=in_specs,
        out_specs=pl.BlockSpec((TB, D, BB), lambda bi, ti: (ti, 0, bi + boff)),
        out_shape=jax.ShapeDtypeStruct((T, D, Bt), jnp.float32),
        input_output_aliases=aliases,
    )


def kernel(inputs, token_table, pos_table):
    Bt, T = inputs.shape
    V, D = token_table.shape
    B = Bt * T
    idx32 = inputs.astype(jnp.int32)
    kh = KW // 2
    idx_rows = jnp.reshape(
        (idx32 // KW) * kh + (idx32 % kh), (B // GW, GW)
    )
    tbl = _tc_transpose(V, D)(jnp.transpose(token_table))
    idxT = jnp.transpose(idx32)
    NH = 2
    Bh = B // NH
    rows_h = Bh // GW
    out_t = None
    for h in range(NH):
        g = _sc_gather(Bh, V // 2, 2 * D)(
            idx_rows[h * rows_h:(h + 1) * rows_h], tbl
        )
        g3 = jnp.reshape(g, (Bt // NH, T, 2 * D))
        fix = _tc_fixup(Bt, T, D, h, NH)
        if h == 0:
            out_t = fix(g3, idxT, pos_table)
        else:
            out_t = fix(g3, idxT, pos_table, out_t)
    return jnp.transpose(out_t, (2, 0, 1))
